# Initial kernel scaffold; baseline (speedup 1.0000x reference)
import jax
import jax.numpy as jnp
from jax.experimental import pallas as pl


def kernel(node_feat, edge_feat, edge_index, W1, b1, W2, b2):
    raise NotImplementedError("write your pallas kernel here")


# speedup vs baseline: 29.2676x; 29.2676x over previous
import functools

import jax
import jax.numpy as jnp
from jax import lax
from jax.experimental import pallas as pl
from jax.experimental.pallas import tpu as pltpu
from jax.experimental.pallas import tpu_sc as plsc

_LANES = 16


def _sc_body(np_pad, e_per_tile, row_chunk,
             dst_hbm, srcp_hbm, node_hbm, counts_hbm, gath_hbm,
             dst_v, hist_v, red_v, cnt_v, idx_v, rows_v, gsem,
             shared_part, shared_tot):
    c = lax.axis_index("c")
    s = lax.axis_index("s")
    zeros_i = jnp.zeros((_LANES,), jnp.int32)
    ones_i = jnp.ones((_LANES,), jnp.int32)
    zeros_f = jnp.zeros((_LANES,), jnp.float32)
    nvec_hist = np_pad // _LANES
    nvec_edge = e_per_tile // _LANES
    cols = np_pad // 16

    @pl.when(c == 0)
    def _hist_and_compact():
        pltpu.sync_copy(dst_hbm.at[pl.ds(s * e_per_tile, e_per_tile)], dst_v)

        def zero_hist(i, carry):
            hist_v[pl.ds(i * _LANES, _LANES)] = zeros_i
            return carry
        lax.fori_loop(0, nvec_hist, zero_hist, 0)

        def scat(i, carry):
            idx = dst_v[pl.ds(i * _LANES, _LANES)]
            plsc.addupdate_scatter(hist_v, [idx], ones_i)
            return carry
        lax.fori_loop(0, nvec_edge, scat, 0)

        pltpu.sync_copy(hist_v, shared_part.at[s])
        plsc.subcore_barrier()
        col0 = s * cols
        for r in range(16):
            pltpu.sync_copy(shared_part.at[r, pl.ds(col0, cols)], red_v.at[r])

        def red(j, carry):
            acc = red_v[0, pl.ds(j * _LANES, _LANES)]
            for r in range(1, 16):
                acc = acc + red_v[r, pl.ds(j * _LANES, _LANES)]
            red_v[0, pl.ds(j * _LANES, _LANES)] = acc
            return carry
        lax.fori_loop(0, cols // _LANES, red, 0)
        pltpu.sync_copy(red_v.at[0], shared_tot.at[pl.ds(col0, cols)])
        plsc.subcore_barrier()

        @pl.when(s == 0)
        def _compact():
            pltpu.sync_copy(shared_tot, hist_v)

            def zero_cnt(i, carry):
                cnt_v[pl.ds(i * _LANES, _LANES)] = zeros_f
                return carry
            lax.fori_loop(0, nvec_hist, zero_cnt, 0)

            def comp(i, off):
                v = hist_v[pl.ds(i * _LANES, _LANES)]
                m = v > 0
                plsc.store_compressed(cnt_v.at[pl.ds(off, _LANES)],
                                      v.astype(jnp.float32), mask=m)
                return off + plsc.all_reduce_population_count(m)[0]
            lax.fori_loop(0, nvec_hist, comp, jnp.int32(0))
            pltpu.sync_copy(cnt_v, counts_hbm)

    @pl.when(c == 1)
    def _gather():
        rows_per_tile = np_pad // 16
        for b in range(rows_per_tile // row_chunk):
            base = s * rows_per_tile + b * row_chunk
            pltpu.sync_copy(srcp_hbm.at[pl.ds(base, row_chunk)], idx_v)
            pltpu.async_copy(node_hbm.at[idx_v], rows_v, gsem).wait()
            pltpu.sync_copy(rows_v, gath_hbm.at[pl.ds(base, row_chunk)])


def _tc_body(counts_ref, gath_ref, ef_ref, w1_ref, b1_ref, w2_ref, b2_ref,
             out_ref):
    cnt = counts_ref[0, 0, :]
    rows = (gath_ref[...] + ef_ref[...]) * cnt[:, None]
    h1 = jnp.dot(rows, w1_ref[...], preferred_element_type=jnp.float32)
    h1 = jnp.maximum(h1 + b1_ref[...], 0.0)
    out_ref[...] = (
        jnp.dot(h1, w2_ref[...], preferred_element_type=jnp.float32)
        + b2_ref[...]
    )


@jax.jit
def kernel(node_feat, edge_feat, edge_index, W1, b1, W2, b2):
    N, D = node_feat.shape
    E = edge_feat.shape[0]
    D2 = W1.shape[1]
    NP = ((N + 255) // 256) * 256
    ROW_CHUNK = (NP // 16) // 2
    E_PER_TILE = E // 16

    dst = edge_index[1]
    src_pad = jnp.pad(edge_index[0, :N], (0, NP - N))

    mesh = plsc.VectorSubcoreMesh(core_axis_name="c", subcore_axis_name="s")
    sc = pl.kernel(
        functools.partial(_sc_body, NP, E_PER_TILE, ROW_CHUNK),
        out_type=(
            jax.ShapeDtypeStruct((NP,), jnp.float32),
            jax.ShapeDtypeStruct((NP, D), jnp.float32),
        ),
        mesh=mesh,
        scratch_types=[
            pltpu.VMEM((E_PER_TILE,), jnp.int32),
            pltpu.VMEM((NP,), jnp.int32),
            pltpu.VMEM((16, NP // 16), jnp.int32),
            pltpu.VMEM((NP,), jnp.float32),
            pltpu.VMEM((ROW_CHUNK,), jnp.int32),
            pltpu.VMEM((ROW_CHUNK, D), jnp.float32),
            pltpu.SemaphoreType.DMA,
            pltpu.VMEM_SHARED((16, NP), jnp.int32),
            pltpu.VMEM_SHARED((NP,), jnp.int32),
        ],
        compiler_params=pltpu.CompilerParams(needs_layout_passes=False),
    )
    counts, gath = sc(dst, src_pad, node_feat)

    B = 128
    grid = (NP // B,)
    h = pl.pallas_call(
        _tc_body,
        grid=grid,
        in_specs=[
            pl.BlockSpec((1, 1, B), lambda i: (i, 0, 0)),
            pl.BlockSpec((B, D), lambda i: (i, 0)),
            pl.BlockSpec((B, D), lambda i: (i, 0)),
            pl.BlockSpec((D, D2), lambda i: (0, 0)),
            pl.BlockSpec((1, D2), lambda i: (0, 0)),
            pl.BlockSpec((D2, D), lambda i: (0, 0)),
            pl.BlockSpec((1, D), lambda i: (0, 0)),
        ],
        out_specs=pl.BlockSpec((B, D), lambda i: (i, 0)),
        out_shape=jax.ShapeDtypeStruct((NP, D), jnp.float32),
    )(counts.reshape(NP // B, 1, B), gath, edge_feat,
      W1, b1.reshape(1, D2), W2, b2.reshape(1, D))
    return h[:N]


# speedup vs baseline: 32.4421x; 1.1085x over previous
import functools

import jax
import jax.numpy as jnp
from jax import lax
from jax.experimental import pallas as pl
from jax.experimental.pallas import tpu as pltpu
from jax.experimental.pallas import tpu_sc as plsc

_LANES = 16


def _sc_body(np_pad, e_per_tile, row_chunk,
             dst_hbm, srcp_hbm, node_hbm, counts_hbm, gath_hbm,
             dst_v, hist_v, red_v, cnt_v, idx_v, rows_v, gsem,
             shared_part, shared_tot):
    c = lax.axis_index("c")
    s = lax.axis_index("s")
    zeros_i = jnp.zeros((_LANES,), jnp.int32)
    ones_i = jnp.ones((_LANES,), jnp.int32)
    zeros_f = jnp.zeros((_LANES,), jnp.float32)
    nvec_hist = np_pad // _LANES
    nvec_edge = e_per_tile // _LANES
    cols = np_pad // 16

    @pl.when(c == 0)
    def _hist_and_compact():
        pltpu.sync_copy(dst_hbm.at[pl.ds(s * e_per_tile, e_per_tile)], dst_v)

        def zero_hist(i, carry):
            hist_v[pl.ds(i * _LANES, _LANES)] = zeros_i
            return carry
        lax.fori_loop(0, nvec_hist, zero_hist, 0, unroll=8)

        def scat(i, carry):
            idx = dst_v[pl.ds(i * _LANES, _LANES)]
            plsc.addupdate_scatter(hist_v, [idx], ones_i)
            return carry
        lax.fori_loop(0, nvec_edge, scat, 0, unroll=8)

        pltpu.sync_copy(hist_v, shared_part.at[s])
        plsc.subcore_barrier()
        col0 = s * cols
        for r in range(16):
            pltpu.sync_copy(shared_part.at[r, pl.ds(col0, cols)], red_v.at[r])

        def red(j, carry):
            acc = red_v[0, pl.ds(j * _LANES, _LANES)]
            for r in range(1, 16):
                acc = acc + red_v[r, pl.ds(j * _LANES, _LANES)]
            red_v[0, pl.ds(j * _LANES, _LANES)] = acc
            return carry
        lax.fori_loop(0, cols // _LANES, red, 0, unroll=4)
        pltpu.sync_copy(red_v.at[0], shared_tot.at[pl.ds(col0, cols)])
        plsc.subcore_barrier()

        @pl.when(s == 0)
        def _compact():
            pltpu.sync_copy(shared_tot, hist_v)

            def zero_cnt(i, carry):
                cnt_v[pl.ds(i * _LANES, _LANES)] = zeros_f
                return carry
            lax.fori_loop(0, nvec_hist, zero_cnt, 0, unroll=8)

            def comp(i, off):
                v = hist_v[pl.ds(i * _LANES, _LANES)]
                m = v > 0
                plsc.store_compressed(cnt_v.at[pl.ds(off, _LANES)],
                                      v.astype(jnp.float32), mask=m)
                return off + plsc.all_reduce_population_count(m)[0]
            lax.fori_loop(0, nvec_hist, comp, jnp.int32(0), unroll=4)
            pltpu.sync_copy(cnt_v, counts_hbm)

    @pl.when(c == 1)
    def _gather():
        rows_per_tile = np_pad // 16
        for b in range(rows_per_tile // row_chunk):
            base = s * rows_per_tile + b * row_chunk
            pltpu.sync_copy(srcp_hbm.at[pl.ds(base, row_chunk)], idx_v)
            pltpu.async_copy(node_hbm.at[idx_v], rows_v, gsem).wait()
            pltpu.sync_copy(rows_v, gath_hbm.at[pl.ds(base, row_chunk)])


def _tc_body(counts_ref, gath_ref, ef_ref, w1_ref, b1_ref, w2_ref, b2_ref,
             out_ref):
    cnt = counts_ref[0, 0, :]
    rows = (gath_ref[...] + ef_ref[...]) * cnt[:, None]
    h1 = jnp.dot(rows, w1_ref[...], preferred_element_type=jnp.float32)
    h1 = jnp.maximum(h1 + b1_ref[...], 0.0)
    out_ref[...] = (
        jnp.dot(h1, w2_ref[...], preferred_element_type=jnp.float32)
        + b2_ref[...]
    )


@jax.jit
def kernel(node_feat, edge_feat, edge_index, W1, b1, W2, b2):
    N, D = node_feat.shape
    E = edge_feat.shape[0]
    D2 = W1.shape[1]
    NP = ((N + 255) // 256) * 256
    ROW_CHUNK = (NP // 16) // 2
    E_PER_TILE = E // 16

    mesh = plsc.VectorSubcoreMesh(core_axis_name="c", subcore_axis_name="s")
    sc = pl.kernel(
        functools.partial(_sc_body, NP, E_PER_TILE, ROW_CHUNK),
        out_type=(
            jax.ShapeDtypeStruct((NP,), jnp.float32),
            jax.ShapeDtypeStruct((NP, D), jnp.float32),
        ),
        mesh=mesh,
        scratch_types=[
            pltpu.VMEM((E_PER_TILE,), jnp.int32),
            pltpu.VMEM((NP,), jnp.int32),
            pltpu.VMEM((16, NP // 16), jnp.int32),
            pltpu.VMEM((NP,), jnp.float32),
            pltpu.VMEM((ROW_CHUNK,), jnp.int32),
            pltpu.VMEM((ROW_CHUNK, D), jnp.float32),
            pltpu.SemaphoreType.DMA,
            pltpu.VMEM_SHARED((16, NP), jnp.int32),
            pltpu.VMEM_SHARED((NP,), jnp.int32),
        ],
        compiler_params=pltpu.CompilerParams(needs_layout_passes=False),
    )
    counts, gath = sc(edge_index[1], edge_index[0, :NP], node_feat)

    B = 128
    grid = (pl.cdiv(N, B),)
    h = pl.pallas_call(
        _tc_body,
        grid=grid,
        in_specs=[
            pl.BlockSpec((1, 1, B), lambda i: (i, 0, 0)),
            pl.BlockSpec((B, D), lambda i: (i, 0)),
            pl.BlockSpec((B, D), lambda i: (i, 0)),
            pl.BlockSpec((D, D2), lambda i: (0, 0)),
            pl.BlockSpec((1, D2), lambda i: (0, 0)),
            pl.BlockSpec((D2, D), lambda i: (0, 0)),
            pl.BlockSpec((1, D), lambda i: (0, 0)),
        ],
        out_specs=pl.BlockSpec((B, D), lambda i: (i, 0)),
        out_shape=jax.ShapeDtypeStruct((N, D), jnp.float32),
    )(counts.reshape(NP // B, 1, B), gath, edge_feat,
      W1, b1.reshape(1, D2), W2, b2.reshape(1, D))
    return h


# speedup vs baseline: 45.2637x; 1.3952x over previous
import functools

import jax
import jax.numpy as jnp
from jax import lax
from jax.experimental import pallas as pl
from jax.experimental.pallas import tpu as pltpu
from jax.experimental.pallas import tpu_sc as plsc

_LANES = 16


def _sc_body(np_pad, e_per_tile, row_chunk,
             dst_hbm, srcp_hbm, node_hbm, counts_hbm, gath_hbm,
             dst_v, hist_v, red_v, cnt_v, idx_v, rows_v, gsem,
             shared_part, shared_tot):
    c = lax.axis_index("c")
    s = lax.axis_index("s")
    zeros_i = jnp.zeros((_LANES,), jnp.int32)
    ones_i = jnp.ones((_LANES,), jnp.int32)
    zeros_f = jnp.zeros((_LANES,), jnp.float32)
    nvec_hist = np_pad // _LANES
    nvec_edge = e_per_tile // _LANES
    cols = np_pad // 16

    @pl.when(c == 0)
    def _hist_and_compact():
        pltpu.sync_copy(dst_hbm.at[pl.ds(s * e_per_tile, e_per_tile)], dst_v)

        def zero_hist(i, carry):
            hist_v[pl.ds(i * _LANES, _LANES)] = zeros_i
            return carry
        lax.fori_loop(0, nvec_hist, zero_hist, 0, unroll=8)

        def scat(i, carry):
            idx = dst_v[pl.ds(i * _LANES, _LANES)]
            plsc.addupdate_scatter(hist_v, [idx], ones_i)
            return carry
        lax.fori_loop(0, nvec_edge, scat, 0, unroll=8)

        pltpu.sync_copy(hist_v, shared_part.at[s])
        plsc.subcore_barrier()
        col0 = s * cols
        for r in range(16):
            pltpu.sync_copy(shared_part.at[r, pl.ds(col0, cols)], red_v.at[r])

        def red(j, carry):
            acc = red_v[0, pl.ds(j * _LANES, _LANES)]
            for r in range(1, 16):
                acc = acc + red_v[r, pl.ds(j * _LANES, _LANES)]
            red_v[0, pl.ds(j * _LANES, _LANES)] = acc
            return carry
        lax.fori_loop(0, cols // _LANES, red, 0, unroll=4)
        pltpu.sync_copy(red_v.at[0], shared_tot.at[pl.ds(col0, cols)])
        plsc.subcore_barrier()

        @pl.when(s == 0)
        def _compact():
            pltpu.sync_copy(shared_tot, hist_v)

            def zero_cnt(i, carry):
                cnt_v[pl.ds(i * _LANES, _LANES)] = zeros_f
                return carry
            lax.fori_loop(0, nvec_hist, zero_cnt, 0, unroll=8)

            def comp(i, off):
                v = hist_v[pl.ds(i * _LANES, _LANES)]
                m = v > 0
                plsc.store_compressed(cnt_v.at[pl.ds(off, _LANES)],
                                      v.astype(jnp.float32), mask=m)
                return off + plsc.all_reduce_population_count(m)[0]
            lax.fori_loop(0, nvec_hist, comp, jnp.int32(0), unroll=4)
            pltpu.sync_copy(cnt_v, counts_hbm)

    @pl.when(c == 1)
    def _gather():
        rows_per_tile = np_pad // 16
        for b in range(rows_per_tile // row_chunk):
            base = s * rows_per_tile + b * row_chunk
            pltpu.sync_copy(srcp_hbm.at[pl.ds(base, row_chunk)], idx_v)
            pltpu.async_copy(node_hbm.at[idx_v], rows_v, gsem).wait()
            pltpu.sync_copy(rows_v, gath_hbm.at[pl.ds(base, row_chunk)])


def _tc_body(counts_ref, gath_ref, ef_ref, w1_ref, b1_ref, w2_ref, b2_ref,
             out_ref):
    rows = (gath_ref[...] + ef_ref[...]) * counts_ref[...]
    h1 = jnp.dot(rows, w1_ref[...], preferred_element_type=jnp.float32)
    h1 = jnp.maximum(h1 + b1_ref[...], 0.0)
    out_ref[...] = (
        jnp.dot(h1, w2_ref[...], preferred_element_type=jnp.float32)
        + b2_ref[...]
    )


@jax.jit
def kernel(node_feat, edge_feat, edge_index, W1, b1, W2, b2):
    N, D = node_feat.shape
    E = edge_feat.shape[0]
    D2 = W1.shape[1]
    NP = ((N + 255) // 256) * 256
    ROW_CHUNK = (NP // 16) // 2
    E_PER_TILE = E // 16

    mesh = plsc.VectorSubcoreMesh(core_axis_name="c", subcore_axis_name="s")
    sc = pl.kernel(
        functools.partial(_sc_body, NP, E_PER_TILE, ROW_CHUNK),
        out_type=(
            jax.ShapeDtypeStruct((NP,), jnp.float32),
            jax.ShapeDtypeStruct((NP, D), jnp.float32),
        ),
        mesh=mesh,
        scratch_types=[
            pltpu.VMEM((E_PER_TILE,), jnp.int32),
            pltpu.VMEM((NP,), jnp.int32),
            pltpu.VMEM((16, NP // 16), jnp.int32),
            pltpu.VMEM((NP,), jnp.float32),
            pltpu.VMEM((ROW_CHUNK,), jnp.int32),
            pltpu.VMEM((ROW_CHUNK, D), jnp.float32),
            pltpu.SemaphoreType.DMA,
            pltpu.VMEM_SHARED((16, NP), jnp.int32),
            pltpu.VMEM_SHARED((NP,), jnp.int32),
        ],
        compiler_params=pltpu.CompilerParams(needs_layout_passes=False),
    )
    counts, gath = sc(edge_index[1], edge_index[0, :NP], node_feat)

    B = 512
    grid = (pl.cdiv(N, B),)
    h = pl.pallas_call(
        _tc_body,
        grid=grid,
        in_specs=[
            pl.BlockSpec((B, 1), lambda i: (i, 0)),
            pl.BlockSpec((B, D), lambda i: (i, 0)),
            pl.BlockSpec((B, D), lambda i: (i, 0)),
            pl.BlockSpec((D, D2), lambda i: (0, 0)),
            pl.BlockSpec((1, D2), lambda i: (0, 0)),
            pl.BlockSpec((D2, D), lambda i: (0, 0)),
            pl.BlockSpec((1, D), lambda i: (0, 0)),
        ],
        out_specs=pl.BlockSpec((B, D), lambda i: (i, 0)),
        out_shape=jax.ShapeDtypeStruct((N, D), jnp.float32),
    )(counts.reshape(NP, 1), gath, edge_feat,
      W1, b1.reshape(1, D2), W2, b2.reshape(1, D))
    return h


# speedup vs baseline: 45.3927x; 1.0028x over previous
import functools

import jax
import jax.numpy as jnp
from jax import lax
from jax.experimental import pallas as pl
from jax.experimental.pallas import tpu as pltpu
from jax.experimental.pallas import tpu_sc as plsc

_LANES = 16


def _sc_body(np_pad, e_per_tile, row_chunk,
             dst_hbm, srcp_hbm, node_hbm, counts_hbm, gath_hbm,
             dst_v, hist_v, red_v, cnt_v, idx_v, rows_v, rows_v2,
             gsem, gsem2, wsem, wsem2,
             shared_part, shared_tot):
    c = lax.axis_index("c")
    s = lax.axis_index("s")
    zeros_i = jnp.zeros((_LANES,), jnp.int32)
    ones_i = jnp.ones((_LANES,), jnp.int32)
    zeros_f = jnp.zeros((_LANES,), jnp.float32)
    nvec_hist = np_pad // _LANES
    nvec_edge = e_per_tile // _LANES
    cols = np_pad // 16

    @pl.when(c == 0)
    def _hist_and_compact():
        dcp = pltpu.async_copy(
            dst_hbm.at[pl.ds(s * e_per_tile, e_per_tile)], dst_v, gsem)

        def zero_hist(i, carry):
            hist_v[pl.ds(i * _LANES, _LANES)] = zeros_i
            return carry
        lax.fori_loop(0, nvec_hist, zero_hist, 0, unroll=8)
        dcp.wait()

        def scat(i, carry):
            idx = dst_v[pl.ds(i * _LANES, _LANES)]
            plsc.addupdate_scatter(hist_v, [idx], ones_i)
            return carry
        lax.fori_loop(0, nvec_edge, scat, 0, unroll=8)

        pltpu.sync_copy(hist_v, shared_part.at[s])
        plsc.subcore_barrier()
        col0 = s * cols
        for r in range(16):
            pltpu.sync_copy(shared_part.at[r, pl.ds(col0, cols)], red_v.at[r])

        def red(j, carry):
            acc = red_v[0, pl.ds(j * _LANES, _LANES)]
            for r in range(1, 16):
                acc = acc + red_v[r, pl.ds(j * _LANES, _LANES)]
            red_v[0, pl.ds(j * _LANES, _LANES)] = acc
            return carry
        lax.fori_loop(0, cols // _LANES, red, 0, unroll=4)
        pltpu.sync_copy(red_v.at[0], shared_tot.at[pl.ds(col0, cols)])
        plsc.subcore_barrier()

        @pl.when(s == 0)
        def _compact():
            pltpu.sync_copy(shared_tot, hist_v)

            def zero_cnt(i, carry):
                cnt_v[pl.ds(i * _LANES, _LANES)] = zeros_f
                return carry
            lax.fori_loop(0, nvec_hist, zero_cnt, 0, unroll=8)

            def comp(i, off):
                v = hist_v[pl.ds(i * _LANES, _LANES)]
                m = v > 0
                plsc.store_compressed(cnt_v.at[pl.ds(off, _LANES)],
                                      v.astype(jnp.float32), mask=m)
                return off + plsc.all_reduce_population_count(m)[0]
            lax.fori_loop(0, nvec_hist, comp, jnp.int32(0), unroll=4)
            pltpu.sync_copy(cnt_v, counts_hbm)

    @pl.when(c == 1)
    def _gather():
        rows_per_tile = np_pad // 16
        nb = rows_per_tile // row_chunk
        base = s * rows_per_tile
        pltpu.sync_copy(srcp_hbm.at[pl.ds(base, rows_per_tile)], idx_v)
        bufs = (rows_v, rows_v2)
        gsems = (gsem, gsem2)
        wsems = (wsem, wsem2)
        gd = [None, None]
        wd = [None, None]

        def gstart(b):
            gd[b % 2] = pltpu.async_copy(
                node_hbm.at[idx_v.at[pl.ds(b * row_chunk, row_chunk)]],
                bufs[b % 2], gsems[b % 2])

        gstart(0)
        for b in range(nb):
            if b + 1 < nb:
                if wd[(b + 1) % 2] is not None:
                    wd[(b + 1) % 2].wait()
                gstart(b + 1)
            gd[b % 2].wait()
            wd[b % 2] = pltpu.async_copy(
                bufs[b % 2],
                gath_hbm.at[pl.ds(base + b * row_chunk, row_chunk)],
                wsems[b % 2])
        for d in wd:
            if d is not None:
                d.wait()


def _tc_body(counts_ref, gath_ref, ef_ref, w1_ref, b1_ref, w2_ref, b2_ref,
             out_ref):
    rows = (gath_ref[...] + ef_ref[...]) * counts_ref[...]
    h1 = jnp.dot(rows, w1_ref[...], preferred_element_type=jnp.float32)
    h1 = jnp.maximum(h1 + b1_ref[...], 0.0)
    out_ref[...] = (
        jnp.dot(h1, w2_ref[...], preferred_element_type=jnp.float32)
        + b2_ref[...]
    )


@jax.jit
def kernel(node_feat, edge_feat, edge_index, W1, b1, W2, b2):
    N, D = node_feat.shape
    E = edge_feat.shape[0]
    D2 = W1.shape[1]
    NP = ((N + 255) // 256) * 256
    ROW_CHUNK = (NP // 16) // 4
    E_PER_TILE = E // 16

    mesh = plsc.VectorSubcoreMesh(core_axis_name="c", subcore_axis_name="s")
    sc = pl.kernel(
        functools.partial(_sc_body, NP, E_PER_TILE, ROW_CHUNK),
        out_type=(
            jax.ShapeDtypeStruct((NP,), jnp.float32),
            jax.ShapeDtypeStruct((NP, D), jnp.float32),
        ),
        mesh=mesh,
        scratch_types=[
            pltpu.VMEM((E_PER_TILE,), jnp.int32),
            pltpu.VMEM((NP,), jnp.int32),
            pltpu.VMEM((16, NP // 16), jnp.int32),
            pltpu.VMEM((NP,), jnp.float32),
            pltpu.VMEM((NP // 16,), jnp.int32),
            pltpu.VMEM((ROW_CHUNK, D), jnp.float32),
            pltpu.VMEM((ROW_CHUNK, D), jnp.float32),
            pltpu.SemaphoreType.DMA,
            pltpu.SemaphoreType.DMA,
            pltpu.SemaphoreType.DMA,
            pltpu.SemaphoreType.DMA,
            pltpu.VMEM_SHARED((16, NP), jnp.int32),
            pltpu.VMEM_SHARED((NP,), jnp.int32),
        ],
        compiler_params=pltpu.CompilerParams(needs_layout_passes=False),
    )
    counts, gath = sc(edge_index[1], edge_index[0, :NP], node_feat)

    B = 512
    grid = (pl.cdiv(N, B),)
    h = pl.pallas_call(
        _tc_body,
        grid=grid,
        in_specs=[
            pl.BlockSpec((B, 1), lambda i: (i, 0)),
            pl.BlockSpec((B, D), lambda i: (i, 0)),
            pl.BlockSpec((B, D), lambda i: (i, 0)),
            pl.BlockSpec((D, D2), lambda i: (0, 0)),
            pl.BlockSpec((1, D2), lambda i: (0, 0)),
            pl.BlockSpec((D2, D), lambda i: (0, 0)),
            pl.BlockSpec((1, D), lambda i: (0, 0)),
        ],
        out_specs=pl.BlockSpec((B, D), lambda i: (i, 0)),
        out_shape=jax.ShapeDtypeStruct((N, D), jnp.float32),
    )(counts.reshape(NP, 1), gath, edge_feat,
      W1, b1.reshape(1, D2), W2, b2.reshape(1, D))
    return h
